# Initial kernel scaffold; baseline (speedup 1.0000x reference)
import jax
import jax.numpy as jnp
from jax.experimental import pallas as pl


def kernel(x1, edge_index1, edge_attr1, x2, edge_index2, edge_attr2, W1, b1, W4, b4, Wc1, bc1, Wc2, bc2):
    raise NotImplementedError("write your pallas kernel here")


# speedup vs baseline: 109.1504x; 109.1504x over previous
import functools

import jax
import jax.numpy as jnp
from jax import lax
from jax.experimental import pallas as pl
from jax.experimental.pallas import tpu as pltpu
from jax.experimental.pallas import tpu_sc as plsc

N = 268
E = 8576
NFLAT = N * N
L = 16


def _sc_build_adjacency(src_all, dst_all, ew_all):
    gtot = src_all.shape[0]
    info = plsc.get_sparse_core_info()
    nc, ns = info.num_cores, info.num_subcores
    nw = nc * ns
    per_w = gtot // nw
    assert per_w * nw == gtot

    mesh = plsc.VectorSubcoreMesh(core_axis_name="c", subcore_axis_name="s")

    @functools.partial(
        pl.kernel,
        mesh=mesh,
        out_type=jax.ShapeDtypeStruct((gtot, NFLAT), jnp.float32),
        scratch_types=[
            pltpu.VMEM((E,), jnp.int32),
            pltpu.VMEM((E,), jnp.int32),
            pltpu.VMEM((E,), jnp.float32),
            pltpu.VMEM((NFLAT,), jnp.float32),
        ],
        compiler_params=pltpu.CompilerParams(needs_layout_passes=False),
    )
    def scatter_kernel(src_hbm, dst_hbm, ew_hbm, out_hbm, src_v, dst_v, ew_v, a_v):
        wid = lax.axis_index("s") * nc + lax.axis_index("c")

        def zero_body(i, carry):
            a_v[pl.ds(i * L, L)] = jnp.zeros((L,), jnp.float32)
            return carry

        def graph_body(gi, carry):
            g = wid * per_w + gi
            pltpu.sync_copy(src_hbm.at[g], src_v)
            pltpu.sync_copy(dst_hbm.at[g], dst_v)
            pltpu.sync_copy(ew_hbm.at[g], ew_v)

            def scat_body(i, c):
                s = src_v[pl.ds(i * L, L)]
                d = dst_v[pl.ds(i * L, L)]
                w = ew_v[pl.ds(i * L, L)]
                plsc.addupdate_scatter(a_v, [d * N + s], w)
                return c

            lax.fori_loop(0, E // L, scat_body, 0, unroll=False)
            pltpu.sync_copy(a_v, out_hbm.at[g])

            def unscat_body(i, c):
                s = src_v[pl.ds(i * L, L)]
                d = dst_v[pl.ds(i * L, L)]
                plsc.store_scatter(a_v, [d * N + s], jnp.zeros((L,), jnp.float32))
                return c

            lax.fori_loop(0, E // L, unscat_body, 0, unroll=False)
            return carry

        lax.fori_loop(0, NFLAT // L, zero_body, 0, unroll=False)
        lax.fori_loop(0, per_w, graph_body, 0, unroll=False)

    return scatter_kernel(src_all, dst_all, ew_all)


def _tc_body(x_ref, a_ref, w1_ref, b1_ref, w4_ref, b4_ref, wc1_ref, bc1_ref,
             wc2_ref, bc2_ref, out_ref):
    f32 = jnp.float32
    hi = lax.Precision.HIGHEST
    x = x_ref[0]
    a = a_ref[0]

    ones_col = jnp.ones((N, 1), f32)
    deg_c = lax.dot_general(a, ones_col, (((0,), (0,)), ((), ())),
                            precision=hi, preferred_element_type=f32)
    deg_r = jnp.sum(a, axis=0, keepdims=True)
    dis_c = jnp.where(deg_c > 0, lax.rsqrt(jnp.where(deg_c > 0, deg_c, 1.0)), 0.0)
    dis_r = jnp.where(deg_r > 0, lax.rsqrt(jnp.where(deg_r > 0, deg_r, 1.0)), 0.0)
    s = -(dis_c * a * dis_r)

    def mm(p, q):
        return lax.dot_general(p, q, (((1,), (0,)), ((), ())),
                               precision=hi, preferred_element_type=f32)

    tx1 = mm(s, x)
    tx2 = 2.0 * mm(s, tx1) - x
    h = mm(x, w1_ref[0]) + mm(tx1, w1_ref[1]) + mm(tx2, w1_ref[2]) + b1_ref[...]
    h = jnp.maximum(h, 0.0)

    u1 = mm(s, h)
    u2 = 2.0 * mm(s, u1) - h
    z = mm(h, w4_ref[0]) + mm(u1, w4_ref[1]) + mm(u2, w4_ref[2]) + b4_ref[...]

    zc = lax.dot_general(z, wc1_ref[...], (((0,), (0,)), ((), ())),
                         precision=hi, preferred_element_type=f32)
    zc = jnp.maximum(zc + bc1_ref[...], 0.0)
    out = mm(zc, wc2_ref[...]) + bc2_ref[...]
    out_ref[0] = out


def _tc_dense(x_all, a_all, w1, b1, w4, b4, wc1, bc1, wc2, bc2, interpret=False):
    gtot = x_all.shape[0]
    nclass = w4.shape[-1]
    nout = wc2.shape[-1]

    full = lambda shape: pl.BlockSpec(shape, lambda g: (0,) * len(shape))
    grid_spec = pl.GridSpec(
        grid=(gtot,),
        in_specs=[
            pl.BlockSpec((1, N, N), lambda g: (g, 0, 0)),
            pl.BlockSpec((1, N, N), lambda g: (g, 0, 0)),
            full(w1.shape),
            full((1, b1.shape[0])),
            full(w4.shape),
            full((1, b4.shape[0])),
            full(wc1.shape),
            full((1, bc1.shape[0])),
            full(wc2.shape),
            full((1, bc2.shape[0])),
        ],
        out_specs=pl.BlockSpec((1, nclass, nout), lambda g: (g, 0, 0)),
    )
    return pl.pallas_call(
        _tc_body,
        grid_spec=grid_spec,
        out_shape=jax.ShapeDtypeStruct((gtot, nclass, nout), jnp.float32),
        interpret=interpret,
    )(x_all, a_all, w1, b1.reshape(1, -1), w4, b4.reshape(1, -1),
      wc1, bc1.reshape(1, -1), wc2, bc2.reshape(1, -1))


def kernel(x1, edge_index1, edge_attr1, x2, edge_index2, edge_attr2,
           W1, b1, W4, b4, Wc1, bc1, Wc2, bc2):
    g = x1.shape[0]
    x_all = jnp.concatenate([x1, x2], axis=0)
    ei = jnp.concatenate([edge_index1, edge_index2], axis=0).astype(jnp.int32)
    ew = jnp.concatenate([edge_attr1, edge_attr2], axis=0).astype(jnp.float32)
    src = ei[:, 0, :]
    dst = ei[:, 1, :]

    a_flat = _sc_build_adjacency(src, dst, ew)
    a_all = a_flat.reshape(-1, N, N)

    out = _tc_dense(x_all, a_all, W1, b1, W4, b4, Wc1, bc1, Wc2, bc2)
    return out[:g], out[g:]


# speedup vs baseline: 161.6833x; 1.4813x over previous
import functools

import jax
import jax.numpy as jnp
from jax import lax
from jax.experimental import pallas as pl
from jax.experimental.pallas import tpu as pltpu
from jax.experimental.pallas import tpu_sc as plsc

N = 268
E = 8576
L = 16
NP = 272


def _sc_build_adjacency(ei1, ea1, ei2, ea2):
    g = ea1.shape[0]
    info = plsc.get_sparse_core_info()
    nc, ns = info.num_cores, info.num_subcores
    assert nc * ns == g

    mesh = plsc.VectorSubcoreMesh(core_axis_name="c", subcore_axis_name="s")

    @functools.partial(
        pl.kernel,
        mesh=mesh,
        out_type=(
            jax.ShapeDtypeStruct((g, N, NP), jnp.float32),
            jax.ShapeDtypeStruct((g, N, NP), jnp.float32),
        ),
        scratch_types=[
            pltpu.VMEM((E,), jnp.int32),
            pltpu.VMEM((E,), jnp.int32),
            pltpu.VMEM((E,), jnp.float32),
            pltpu.VMEM((N, NP), jnp.float32),
        ],
        compiler_params=pltpu.CompilerParams(needs_layout_passes=False),
    )
    def scatter_kernel(ei1_h, ea1_h, ei2_h, ea2_h, o1_h, o2_h,
                       src_v, dst_v, ew_v, a_v):
        w = lax.axis_index("s") * nc + lax.axis_index("c")
        zeros16 = jnp.zeros((L,), jnp.float32)

        def zero_row(i, c):
            def zero_col(j, c2):
                a_v[i, pl.ds(j * L, L)] = zeros16
                return c2

            lax.fori_loop(0, NP // L, zero_col, 0, unroll=False)
            return c

        lax.fori_loop(0, N, zero_row, 0, unroll=False)

        def scat(i, c):
            s = src_v[pl.ds(i * L, L)]
            d = dst_v[pl.ds(i * L, L)]
            v = ew_v[pl.ds(i * L, L)]
            plsc.addupdate_scatter(a_v, [d, s], v)
            return c

        def unscat(i, c):
            s = src_v[pl.ds(i * L, L)]
            d = dst_v[pl.ds(i * L, L)]
            plsc.store_scatter(a_v, [d, s], zeros16)
            return c

        for ei_h, ea_h, o_h, last in (
            (ei1_h, ea1_h, o1_h, False),
            (ei2_h, ea2_h, o2_h, True),
        ):
            pltpu.sync_copy(ei_h.at[2 * w], src_v)
            pltpu.sync_copy(ei_h.at[2 * w + 1], dst_v)
            pltpu.sync_copy(ea_h.at[w], ew_v)
            lax.fori_loop(0, E // L, scat, 0, unroll=False)
            pltpu.sync_copy(a_v, o_h.at[w])
            if not last:
                lax.fori_loop(0, E // L, unscat, 0, unroll=False)

    return scatter_kernel(ei1, ea1, ei2, ea2)


def _tc_body(x_ref, a_ref, w1_ref, b1_ref, w4_ref, b4_ref, wc1_ref, bc1_ref,
             wc2_ref, bc2_ref, out_ref):
    f32 = jnp.float32
    hi = lax.Precision.HIGHEST
    x = x_ref[0]
    a = a_ref[0][:, :N]

    def mm(p, q):
        return lax.dot_general(p, q, (((1,), (0,)), ((), ())),
                               precision=hi, preferred_element_type=f32)

    deg_r = jnp.sum(a, axis=0, keepdims=True)
    dis_r = jnp.where(deg_r > 0, lax.rsqrt(jnp.where(deg_r > 0, deg_r, 1.0)),
                      0.0)
    dis_c = jnp.transpose(dis_r)
    b_mat = a * dis_r

    p = mm(x, w1_ref[...])
    sp2 = dis_c * mm(b_mat, p[:, 64:96])
    sall = dis_c * mm(b_mat, p[:, 32:64] - 2.0 * sp2)
    h = jnp.maximum(p[:, 0:32] - sall + b1_ref[...], 0.0)

    q = mm(h, w4_ref[...])
    sq2 = dis_c * mm(b_mat, q[:, 64:96])
    sall2 = dis_c * mm(b_mat, q[:, 32:64] - 2.0 * sq2)
    z = q[:, 0:32] - sall2 + b4_ref[...]

    zc = lax.dot_general(z, wc1_ref[...], (((0,), (0,)), ((), ())),
                         precision=hi, preferred_element_type=f32)
    zc = jnp.maximum(zc + bc1_ref[...], 0.0)
    out = mm(zc, wc2_ref[...]) + bc2_ref[...]
    out_ref[0] = out


def _tc_dense(x_all, a_all, w1c, b1, w4c, b4, wc1, bc1, wc2, bc2,
              interpret=False):
    g = x_all.shape[0]
    nclass = 32
    nout = wc2.shape[-1]

    full = lambda shape: pl.BlockSpec(shape, lambda i: (0,) * len(shape))
    grid_spec = pl.GridSpec(
        grid=(g,),
        in_specs=[
            pl.BlockSpec((1, N, N), lambda i: (i, 0, 0)),
            pl.BlockSpec((1, N, NP), lambda i: (i, 0, 0)),
            full(w1c.shape),
            full((1, b1.shape[0])),
            full(w4c.shape),
            full((1, b4.shape[0])),
            full(wc1.shape),
            full((1, bc1.shape[0])),
            full(wc2.shape),
            full((1, bc2.shape[0])),
        ],
        out_specs=pl.BlockSpec((1, nclass, nout), lambda i: (i, 0, 0)),
    )
    return pl.pallas_call(
        _tc_body,
        grid_spec=grid_spec,
        out_shape=jax.ShapeDtypeStruct((g, nclass, nout), jnp.float32),
        interpret=interpret,
    )(x_all, a_all, w1c, b1.reshape(1, -1), w4c, b4.reshape(1, -1),
      wc1, bc1.reshape(1, -1), wc2, bc2.reshape(1, -1))


def kernel(x1, edge_index1, edge_attr1, x2, edge_index2, edge_attr2,
           W1, b1, W4, b4, Wc1, bc1, Wc2, bc2):
    g = x1.shape[0]
    ei1 = edge_index1.astype(jnp.int32).reshape(2 * g, E)
    ei2 = edge_index2.astype(jnp.int32).reshape(2 * g, E)
    ea1 = edge_attr1.astype(jnp.float32)
    ea2 = edge_attr2.astype(jnp.float32)

    a1, a2 = _sc_build_adjacency(ei1, ea1, ei2, ea2)

    w1c = jnp.concatenate([W1[0] - W1[2], W1[1], W1[2]], axis=1)
    w4c = jnp.concatenate([W4[0] - W4[2], W4[1], W4[2]], axis=1)

    out1 = _tc_dense(x1, a1, w1c, b1, w4c, b4, Wc1, bc1, Wc2, bc2)
    out2 = _tc_dense(x2, a2, w1c, b1, w4c, b4, Wc1, bc1, Wc2, bc2)
    return out1, out2


# speedup vs baseline: 275.1813x; 1.7020x over previous
import functools

import jax
import jax.numpy as jnp
from jax import lax
from jax.experimental import pallas as pl
from jax.experimental.pallas import tpu as pltpu
from jax.experimental.pallas import tpu_sc as plsc

N = 268
E = 8576
L = 16
NP = 272


def _sc_build_adjacency(ei1, ea1, ei2, ea2):
    g = ea1.shape[0]
    info = plsc.get_sparse_core_info()
    nc, ns = info.num_cores, info.num_subcores
    assert nc * ns == g

    mesh = plsc.VectorSubcoreMesh(core_axis_name="c", subcore_axis_name="s")

    @functools.partial(
        pl.kernel,
        mesh=mesh,
        out_type=(
            jax.ShapeDtypeStruct((g, N, NP), jnp.float32),
            jax.ShapeDtypeStruct((g, N, NP), jnp.float32),
        ),
        scratch_types=[
            pltpu.VMEM((E,), jnp.int32),
            pltpu.VMEM((E,), jnp.int32),
            pltpu.VMEM((E,), jnp.float32),
            pltpu.VMEM((N, NP), jnp.float32),
        ],
        compiler_params=pltpu.CompilerParams(needs_layout_passes=False),
    )
    def scatter_kernel(ei1_h, ea1_h, ei2_h, ea2_h, o1_h, o2_h,
                       src_v, dst_v, ew_v, a_v):
        w = lax.axis_index("s") * nc + lax.axis_index("c")
        zeros16 = jnp.zeros((L,), jnp.float32)

        def zero_row(i, c):
            def zero_col(j, c2):
                a_v[i, pl.ds(j * L, L)] = zeros16
                return c2

            lax.fori_loop(0, NP // L, zero_col, 0, unroll=False)
            return c

        lax.fori_loop(0, N, zero_row, 0, unroll=False)

        def scat(i, c):
            s = src_v[pl.ds(i * L, L)]
            d = dst_v[pl.ds(i * L, L)]
            v = ew_v[pl.ds(i * L, L)]
            plsc.addupdate_scatter(a_v, [d, s], v)
            return c

        def unscat(i, c):
            s = src_v[pl.ds(i * L, L)]
            d = dst_v[pl.ds(i * L, L)]
            plsc.store_scatter(a_v, [d, s], zeros16)
            return c

        for ei_h, ea_h, o_h, last in (
            (ei1_h, ea1_h, o1_h, False),
            (ei2_h, ea2_h, o2_h, True),
        ):
            pltpu.sync_copy(ei_h.at[2 * w], src_v)
            pltpu.sync_copy(ei_h.at[2 * w + 1], dst_v)
            pltpu.sync_copy(ea_h.at[w], ew_v)
            lax.fori_loop(0, E // L, scat, 0, unroll=False)
            pltpu.sync_copy(a_v, o_h.at[w])
            if not last:
                lax.fori_loop(0, E // L, unscat, 0, unroll=False)

    return scatter_kernel(ei1, ea1, ei2, ea2)


def _tc_body(x_ref, a_ref, w1_ref, b1_ref, w4_ref, b4_ref, wc1_ref, bc1_ref,
             wc2_ref, bc2_ref, out_ref):
    f32 = jnp.float32
    hi = lax.Precision.DEFAULT

    def mm(p, q):
        return lax.dot_general(p, q, (((1,), (0,)), ((), ())),
                               precision=hi, preferred_element_type=f32)

    for k in range(x_ref.shape[0]):
        x = x_ref[k]
        a = a_ref[k][:, :N]

        deg_r = jnp.sum(a, axis=0, keepdims=True)
        dis_r = jnp.where(deg_r > 0,
                          lax.rsqrt(jnp.where(deg_r > 0, deg_r, 1.0)), 0.0)
        dis_c = jnp.transpose(dis_r)
        b_mat = a * dis_r

        p = mm(x, w1_ref[...])
        sp2 = dis_c * mm(b_mat, p[:, 64:96])
        sall = dis_c * mm(b_mat, p[:, 32:64] - 2.0 * sp2)
        h = jnp.maximum(p[:, 0:32] - sall + b1_ref[...], 0.0)

        q = mm(h, w4_ref[...])
        sq2 = dis_c * mm(b_mat, q[:, 64:96])
        sall2 = dis_c * mm(b_mat, q[:, 32:64] - 2.0 * sq2)
        z = q[:, 0:32] - sall2 + b4_ref[...]

        zc = lax.dot_general(z, wc1_ref[...], (((0,), (0,)), ((), ())),
                             precision=hi, preferred_element_type=f32)
        zc = jnp.maximum(zc + bc1_ref[...], 0.0)
        out = mm(zc, wc2_ref[...]) + bc2_ref[...]
        out_ref[k] = out


def _tc_dense(x_all, a_all, w1c, b1, w4c, b4, wc1, bc1, wc2, bc2,
              interpret=False):
    g = x_all.shape[0]
    nclass = 32
    nout = wc2.shape[-1]

    gb = 2
    full = lambda shape: pl.BlockSpec(shape, lambda i: (0,) * len(shape))
    grid_spec = pl.GridSpec(
        grid=(g // gb,),
        in_specs=[
            pl.BlockSpec((gb, N, N), lambda i: (i, 0, 0)),
            pl.BlockSpec((gb, N, NP), lambda i: (i, 0, 0)),
            full(w1c.shape),
            full((1, b1.shape[0])),
            full(w4c.shape),
            full((1, b4.shape[0])),
            full(wc1.shape),
            full((1, bc1.shape[0])),
            full(wc2.shape),
            full((1, bc2.shape[0])),
        ],
        out_specs=pl.BlockSpec((gb, nclass, nout), lambda i: (i, 0, 0)),
    )
    return pl.pallas_call(
        _tc_body,
        grid_spec=grid_spec,
        out_shape=jax.ShapeDtypeStruct((g, nclass, nout), jnp.float32),
        interpret=interpret,
    )(x_all, a_all, w1c, b1.reshape(1, -1), w4c, b4.reshape(1, -1),
      wc1, bc1.reshape(1, -1), wc2, bc2.reshape(1, -1))


def kernel(x1, edge_index1, edge_attr1, x2, edge_index2, edge_attr2,
           W1, b1, W4, b4, Wc1, bc1, Wc2, bc2):
    g = x1.shape[0]
    ei1 = edge_index1.astype(jnp.int32).reshape(2 * g, E)
    ei2 = edge_index2.astype(jnp.int32).reshape(2 * g, E)
    ea1 = edge_attr1.astype(jnp.float32)
    ea2 = edge_attr2.astype(jnp.float32)

    a1, a2 = _sc_build_adjacency(ei1, ea1, ei2, ea2)

    w1c = jnp.concatenate([W1[0] - W1[2], W1[1], W1[2]], axis=1)
    w4c = jnp.concatenate([W4[0] - W4[2], W4[1], W4[2]], axis=1)

    out1 = _tc_dense(x1, a1, w1c, b1, w4c, b4, Wc1, bc1, Wc2, bc2)
    out2 = _tc_dense(x2, a2, w1c, b1, w4c, b4, Wc1, bc1, Wc2, bc2)
    return out1, out2


# speedup vs baseline: 279.3321x; 1.0151x over previous
import functools

import jax
import jax.numpy as jnp
from jax import lax
from jax.experimental import pallas as pl
from jax.experimental.pallas import tpu as pltpu
from jax.experimental.pallas import tpu_sc as plsc

N = 268
E = 8576
L = 16
NP = 272


def _sc_build_adjacency(ei1, ea1, ei2, ea2):
    g = ea1.shape[0]
    info = plsc.get_sparse_core_info()
    nc, ns = info.num_cores, info.num_subcores
    assert nc * ns == g

    mesh = plsc.VectorSubcoreMesh(core_axis_name="c", subcore_axis_name="s")

    @functools.partial(
        pl.kernel,
        mesh=mesh,
        out_type=(
            jax.ShapeDtypeStruct((g, N, NP), jnp.float32),
            jax.ShapeDtypeStruct((g, N, NP), jnp.float32),
        ),
        scratch_types=[
            pltpu.VMEM((E,), jnp.int32),
            pltpu.VMEM((E,), jnp.int32),
            pltpu.VMEM((E,), jnp.float32),
            pltpu.VMEM((N, NP), jnp.float32),
        ],
        compiler_params=pltpu.CompilerParams(needs_layout_passes=False),
    )
    def scatter_kernel(ei1_h, ea1_h, ei2_h, ea2_h, o1_h, o2_h,
                       src_v, dst_v, ew_v, a_v):
        w = lax.axis_index("s") * nc + lax.axis_index("c")
        zeros16 = jnp.zeros((L,), jnp.float32)

        def zero_row(i, c):
            def zero_col(j, c2):
                a_v[i, pl.ds(j * L, L)] = zeros16
                return c2

            lax.fori_loop(0, NP // L, zero_col, 0, unroll=False)
            return c

        lax.fori_loop(0, N, zero_row, 0, unroll=False)

        def scat(i, c):
            s = src_v[pl.ds(i * L, L)]
            d = dst_v[pl.ds(i * L, L)]
            v = ew_v[pl.ds(i * L, L)]
            plsc.addupdate_scatter(a_v, [d, s], v)
            return c

        def unscat(i, c):
            s = src_v[pl.ds(i * L, L)]
            d = dst_v[pl.ds(i * L, L)]
            plsc.store_scatter(a_v, [d, s], zeros16)
            return c

        for ei_h, ea_h, o_h, last in (
            (ei1_h, ea1_h, o1_h, False),
            (ei2_h, ea2_h, o2_h, True),
        ):
            pltpu.sync_copy(ei_h.at[2 * w], src_v)
            pltpu.sync_copy(ei_h.at[2 * w + 1], dst_v)
            pltpu.sync_copy(ea_h.at[w], ew_v)
            lax.fori_loop(0, E // L, scat, 0, unroll=False)
            pltpu.sync_copy(a_v, o_h.at[w])
            if not last:
                lax.fori_loop(0, E // L, unscat, 0, unroll=False)

    return scatter_kernel(ei1, ea1, ei2, ea2)


def _tc_body(x_ref, a_ref, w1_ref, b1_ref, w4_ref, b4_ref, wc1_ref, bc1_ref,
             wc2_ref, bc2_ref, out_ref):
    f32 = jnp.float32
    hi = lax.Precision.DEFAULT

    def mm(p, q):
        return lax.dot_general(p, q, (((1,), (0,)), ((), ())),
                               precision=hi, preferred_element_type=f32)

    for k in range(x_ref.shape[0]):
        x = x_ref[k]
        a = a_ref[k][:, :N]

        deg_r = jnp.sum(a, axis=0, keepdims=True)
        dis_r = jnp.where(deg_r > 0,
                          lax.rsqrt(jnp.where(deg_r > 0, deg_r, 1.0)), 0.0)
        dis_c = jnp.transpose(dis_r)
        b_mat = a * dis_r

        p = mm(x, w1_ref[...])
        sp2 = dis_c * mm(b_mat, p[:, 64:96])
        sall = dis_c * mm(b_mat, p[:, 32:64] - 2.0 * sp2)
        h = jnp.maximum(p[:, 0:32] - sall + b1_ref[...], 0.0)

        q = mm(h, w4_ref[...])
        sq2 = dis_c * mm(b_mat, q[:, 64:96])
        sall2 = dis_c * mm(b_mat, q[:, 32:64] - 2.0 * sq2)
        z = q[:, 0:32] - sall2 + b4_ref[...]

        zc = lax.dot_general(z, wc1_ref[...], (((0,), (0,)), ((), ())),
                             precision=hi, preferred_element_type=f32)
        zc = jnp.maximum(zc + bc1_ref[...], 0.0)
        out = mm(zc, wc2_ref[...]) + bc2_ref[...]
        out_ref[k] = out


def _tc_dense(x_all, a_all, w1c, b1, w4c, b4, wc1, bc1, wc2, bc2,
              interpret=False):
    g = x_all.shape[0]
    nclass = 32
    nout = wc2.shape[-1]

    gb = 4
    full = lambda shape: pl.BlockSpec(shape, lambda i: (0,) * len(shape))
    grid_spec = pl.GridSpec(
        grid=(g // gb,),
        in_specs=[
            pl.BlockSpec((gb, N, N), lambda i: (i, 0, 0)),
            pl.BlockSpec((gb, N, NP), lambda i: (i, 0, 0)),
            full(w1c.shape),
            full((1, b1.shape[0])),
            full(w4c.shape),
            full((1, b4.shape[0])),
            full(wc1.shape),
            full((1, bc1.shape[0])),
            full(wc2.shape),
            full((1, bc2.shape[0])),
        ],
        out_specs=pl.BlockSpec((gb, nclass, nout), lambda i: (i, 0, 0)),
    )
    return pl.pallas_call(
        _tc_body,
        grid_spec=grid_spec,
        out_shape=jax.ShapeDtypeStruct((g, nclass, nout), jnp.float32),
        interpret=interpret,
    )(x_all, a_all, w1c, b1.reshape(1, -1), w4c, b4.reshape(1, -1),
      wc1, bc1.reshape(1, -1), wc2, bc2.reshape(1, -1))


def kernel(x1, edge_index1, edge_attr1, x2, edge_index2, edge_attr2,
           W1, b1, W4, b4, Wc1, bc1, Wc2, bc2):
    g = x1.shape[0]
    ei1 = edge_index1.astype(jnp.int32).reshape(2 * g, E)
    ei2 = edge_index2.astype(jnp.int32).reshape(2 * g, E)
    ea1 = edge_attr1.astype(jnp.float32)
    ea2 = edge_attr2.astype(jnp.float32)

    a1, a2 = _sc_build_adjacency(ei1, ea1, ei2, ea2)

    w1c = jnp.concatenate([W1[0] - W1[2], W1[1], W1[2]], axis=1)
    w4c = jnp.concatenate([W4[0] - W4[2], W4[1], W4[2]], axis=1)

    out1 = _tc_dense(x1, a1, w1c, b1, w4c, b4, Wc1, bc1, Wc2, bc2)
    out2 = _tc_dense(x2, a2, w1c, b1, w4c, b4, Wc1, bc1, Wc2, bc2)
    return out1, out2


# speedup vs baseline: 280.9057x; 1.0056x over previous
import functools

import jax
import jax.numpy as jnp
from jax import lax
from jax.experimental import pallas as pl
from jax.experimental.pallas import tpu as pltpu
from jax.experimental.pallas import tpu_sc as plsc

N = 268
E = 8576
L = 16
NP = 272


def _sc_build_adjacency(ei1, ea1, ei2, ea2):
    g = ea1.shape[0]
    info = plsc.get_sparse_core_info()
    nc, ns = info.num_cores, info.num_subcores
    assert nc * ns == g

    mesh = plsc.VectorSubcoreMesh(core_axis_name="c", subcore_axis_name="s")

    @functools.partial(
        pl.kernel,
        mesh=mesh,
        out_type=(
            jax.ShapeDtypeStruct((g, N, NP), jnp.float32),
            jax.ShapeDtypeStruct((g, N, NP), jnp.float32),
        ),
        scratch_types=[
            pltpu.VMEM((E,), jnp.int32),
            pltpu.VMEM((E,), jnp.int32),
            pltpu.VMEM((E,), jnp.float32),
            pltpu.VMEM((N, NP), jnp.float32),
        ],
        compiler_params=pltpu.CompilerParams(needs_layout_passes=False),
    )
    def scatter_kernel(ei1_h, ea1_h, ei2_h, ea2_h, o1_h, o2_h,
                       src_v, dst_v, ew_v, a_v):
        w = lax.axis_index("s") * nc + lax.axis_index("c")
        zeros16 = jnp.zeros((L,), jnp.float32)

        @plsc.parallel_loop(0, N)
        def _zero(i):
            for j in range(NP // L):
                a_v[i, pl.ds(j * L, L)] = zeros16

        for ei_h, ea_h, o_h, last in (
            (ei1_h, ea1_h, o1_h, False),
            (ei2_h, ea2_h, o2_h, True),
        ):
            pltpu.sync_copy(ei_h.at[2 * w], src_v)
            pltpu.sync_copy(ei_h.at[2 * w + 1], dst_v)
            pltpu.sync_copy(ea_h.at[w], ew_v)

            @plsc.parallel_loop(0, E // L, unroll=4)
            def _scat(i):
                s = src_v[pl.ds(i * L, L)]
                d = dst_v[pl.ds(i * L, L)]
                v = ew_v[pl.ds(i * L, L)]
                plsc.addupdate_scatter(a_v, [d, s], v)

            pltpu.sync_copy(a_v, o_h.at[w])
            if not last:
                @plsc.parallel_loop(0, E // L, unroll=4)
                def _unscat(i):
                    s = src_v[pl.ds(i * L, L)]
                    d = dst_v[pl.ds(i * L, L)]
                    plsc.store_scatter(a_v, [d, s], zeros16)

    return scatter_kernel(ei1, ea1, ei2, ea2)


def _tc_body(x_ref, a_ref, w1_ref, b1_ref, w4_ref, b4_ref, wc1_ref, bc1_ref,
             wc2_ref, bc2_ref, out_ref):
    f32 = jnp.float32
    hi = lax.Precision.DEFAULT

    def mm(p, q):
        return lax.dot_general(p, q, (((1,), (0,)), ((), ())),
                               precision=hi, preferred_element_type=f32)

    for k in range(x_ref.shape[0]):
        x = x_ref[k]
        a = a_ref[k][:, :N]

        deg_r = jnp.sum(a, axis=0, keepdims=True)
        dis_r = jnp.where(deg_r > 0,
                          lax.rsqrt(jnp.where(deg_r > 0, deg_r, 1.0)), 0.0)
        dis_c = jnp.transpose(dis_r)
        b_mat = a * dis_r

        p = mm(x, w1_ref[...])
        sp2 = dis_c * mm(b_mat, p[:, 64:96])
        sall = dis_c * mm(b_mat, p[:, 32:64] - 2.0 * sp2)
        h = jnp.maximum(p[:, 0:32] - sall + b1_ref[...], 0.0)

        q = mm(h, w4_ref[...])
        sq2 = dis_c * mm(b_mat, q[:, 64:96])
        sall2 = dis_c * mm(b_mat, q[:, 32:64] - 2.0 * sq2)
        z = q[:, 0:32] - sall2 + b4_ref[...]

        zc = lax.dot_general(z, wc1_ref[...], (((0,), (0,)), ((), ())),
                             precision=hi, preferred_element_type=f32)
        zc = jnp.maximum(zc + bc1_ref[...], 0.0)
        out = mm(zc, wc2_ref[...]) + bc2_ref[...]
        out_ref[k] = out


def _tc_dense(x_all, a_all, w1c, b1, w4c, b4, wc1, bc1, wc2, bc2,
              interpret=False):
    g = x_all.shape[0]
    nclass = 32
    nout = wc2.shape[-1]

    gb = 4
    full = lambda shape: pl.BlockSpec(shape, lambda i: (0,) * len(shape))
    grid_spec = pl.GridSpec(
        grid=(g // gb,),
        in_specs=[
            pl.BlockSpec((gb, N, N), lambda i: (i, 0, 0)),
            pl.BlockSpec((gb, N, NP), lambda i: (i, 0, 0)),
            full(w1c.shape),
            full((1, b1.shape[0])),
            full(w4c.shape),
            full((1, b4.shape[0])),
            full(wc1.shape),
            full((1, bc1.shape[0])),
            full(wc2.shape),
            full((1, bc2.shape[0])),
        ],
        out_specs=pl.BlockSpec((gb, nclass, nout), lambda i: (i, 0, 0)),
    )
    return pl.pallas_call(
        _tc_body,
        grid_spec=grid_spec,
        out_shape=jax.ShapeDtypeStruct((g, nclass, nout), jnp.float32),
        interpret=interpret,
    )(x_all, a_all, w1c, b1.reshape(1, -1), w4c, b4.reshape(1, -1),
      wc1, bc1.reshape(1, -1), wc2, bc2.reshape(1, -1))


def kernel(x1, edge_index1, edge_attr1, x2, edge_index2, edge_attr2,
           W1, b1, W4, b4, Wc1, bc1, Wc2, bc2):
    g = x1.shape[0]
    ei1 = edge_index1.astype(jnp.int32).reshape(2 * g, E)
    ei2 = edge_index2.astype(jnp.int32).reshape(2 * g, E)
    ea1 = edge_attr1.astype(jnp.float32)
    ea2 = edge_attr2.astype(jnp.float32)

    a1, a2 = _sc_build_adjacency(ei1, ea1, ei2, ea2)

    w1c = jnp.concatenate([W1[0] - W1[2], W1[1], W1[2]], axis=1)
    w4c = jnp.concatenate([W4[0] - W4[2], W4[1], W4[2]], axis=1)

    out1 = _tc_dense(x1, a1, w1c, b1, w4c, b4, Wc1, bc1, Wc2, bc2)
    out2 = _tc_dense(x2, a2, w1c, b1, w4c, b4, Wc1, bc1, Wc2, bc2)
    return out1, out2


# speedup vs baseline: 466.2518x; 1.6598x over previous
import functools

import jax
import jax.numpy as jnp
from jax import lax
from jax.experimental import pallas as pl
from jax.experimental.pallas import tpu as pltpu
from jax.experimental.pallas import tpu_sc as plsc

N = 268
E = 8576
L = 16
NP = 272


def _sc_build_adjacency(ei1, ea1, ei2, ea2):
    g = ea1.shape[0]
    info = plsc.get_sparse_core_info()
    nc, ns = info.num_cores, info.num_subcores
    assert nc * ns == g

    mesh = plsc.VectorSubcoreMesh(core_axis_name="c", subcore_axis_name="s")

    @functools.partial(
        pl.kernel,
        mesh=mesh,
        out_type=(
            jax.ShapeDtypeStruct((g, N, NP), jnp.float32),
            jax.ShapeDtypeStruct((g, N, NP), jnp.float32),
        ),
        scratch_types=[
            pltpu.VMEM((E,), jnp.int32),
            pltpu.VMEM((E,), jnp.int32),
            pltpu.VMEM((E,), jnp.float32),
            pltpu.VMEM((N, NP), jnp.float32),
        ],
        compiler_params=pltpu.CompilerParams(needs_layout_passes=False),
    )
    def scatter_kernel(ei1_h, ea1_h, ei2_h, ea2_h, o1_h, o2_h,
                       src_v, dst_v, ew_v, a_v):
        w = lax.axis_index("s") * nc + lax.axis_index("c")
        zeros16 = jnp.zeros((L,), jnp.float32)

        @plsc.parallel_loop(0, N)
        def _zero(i):
            for j in range(NP // L):
                a_v[i, pl.ds(j * L, L)] = zeros16

        for ei_h, ea_h, o_h, last in (
            (ei1_h, ea1_h, o1_h, False),
            (ei2_h, ea2_h, o2_h, True),
        ):
            pltpu.sync_copy(ei_h.at[2 * w], src_v)
            pltpu.sync_copy(ei_h.at[2 * w + 1], dst_v)
            pltpu.sync_copy(ea_h.at[w], ew_v)

            @plsc.parallel_loop(0, E // L, unroll=4)
            def _scat(i):
                s = src_v[pl.ds(i * L, L)]
                d = dst_v[pl.ds(i * L, L)]
                v = ew_v[pl.ds(i * L, L)]
                plsc.addupdate_scatter(a_v, [d, s], v)

            pltpu.sync_copy(a_v, o_h.at[w])
            if not last:
                @plsc.parallel_loop(0, E // L, unroll=4)
                def _unscat(i):
                    s = src_v[pl.ds(i * L, L)]
                    d = dst_v[pl.ds(i * L, L)]
                    plsc.store_scatter(a_v, [d, s], zeros16)

    return scatter_kernel(ei1, ea1, ei2, ea2)


def _tc_body(x_ref, a_ref, w1_ref, b1_ref, w4_ref, b4_ref, wc1_ref, bc1_ref,
             wc2_ref, bc2_ref, out_ref):
    f32 = jnp.float32
    hi = lax.Precision.DEFAULT

    def mm(p, q):
        return lax.dot_general(p, q, (((1,), (0,)), ((), ())),
                               precision=hi, preferred_element_type=f32)

    gb = x_ref.shape[0]
    ks = range(gb)
    xs = [x_ref[k] for k in ks]
    avs = [a_ref[k][:, :N] for k in ks]

    s_mats = []
    for k in ks:
        deg_r = jnp.sum(avs[k], axis=0, keepdims=True)
        dis_r = jnp.where(deg_r > 0,
                          lax.rsqrt(jnp.where(deg_r > 0, deg_r, 1.0)), 0.0)
        dis_c = jnp.transpose(dis_r)
        s_mats.append((dis_c * avs[k]) * dis_r)

    p = [mm(xs[k], w1_ref[...]) for k in ks]
    sp2 = [mm(s_mats[k], p[k][:, 64:96]) for k in ks]
    sall = [mm(s_mats[k], p[k][:, 32:64] - 2.0 * sp2[k]) for k in ks]
    h = [jnp.maximum(p[k][:, 0:32] - sall[k] + b1_ref[...], 0.0) for k in ks]

    q = [mm(h[k], w4_ref[...]) for k in ks]
    sq2 = [mm(s_mats[k], q[k][:, 64:96]) for k in ks]
    sall2 = [mm(s_mats[k], q[k][:, 32:64] - 2.0 * sq2[k]) for k in ks]
    z = [q[k][:, 0:32] - sall2[k] + b4_ref[...] for k in ks]

    zc = [lax.dot_general(z[k], wc1_ref[...], (((0,), (0,)), ((), ())),
                          precision=hi, preferred_element_type=f32)
          for k in ks]
    zc = [jnp.maximum(zc[k] + bc1_ref[...], 0.0) for k in ks]
    for k in ks:
        out_ref[k] = mm(zc[k], wc2_ref[...]) + bc2_ref[...]


def _tc_dense(x_all, a_all, w1c, b1, w4c, b4, wc1, bc1, wc2, bc2,
              interpret=False):
    g = x_all.shape[0]
    nclass = 32
    nout = wc2.shape[-1]

    gb = 4
    full = lambda shape: pl.BlockSpec(shape, lambda i: (0,) * len(shape))
    grid_spec = pl.GridSpec(
        grid=(g // gb,),
        in_specs=[
            pl.BlockSpec((gb, N, N), lambda i: (i, 0, 0)),
            pl.BlockSpec((gb, N, NP), lambda i: (i, 0, 0)),
            full(w1c.shape),
            full((1, b1.shape[0])),
            full(w4c.shape),
            full((1, b4.shape[0])),
            full(wc1.shape),
            full((1, bc1.shape[0])),
            full(wc2.shape),
            full((1, bc2.shape[0])),
        ],
        out_specs=pl.BlockSpec((gb, nclass, nout), lambda i: (i, 0, 0)),
    )
    return pl.pallas_call(
        _tc_body,
        grid_spec=grid_spec,
        out_shape=jax.ShapeDtypeStruct((g, nclass, nout), jnp.float32),
        interpret=interpret,
    )(x_all, a_all, w1c, b1.reshape(1, -1), w4c, b4.reshape(1, -1),
      wc1, bc1.reshape(1, -1), wc2, bc2.reshape(1, -1))


def kernel(x1, edge_index1, edge_attr1, x2, edge_index2, edge_attr2,
           W1, b1, W4, b4, Wc1, bc1, Wc2, bc2):
    g = x1.shape[0]
    ei1 = edge_index1.astype(jnp.int32).reshape(2 * g, E)
    ei2 = edge_index2.astype(jnp.int32).reshape(2 * g, E)
    ea1 = edge_attr1.astype(jnp.float32)
    ea2 = edge_attr2.astype(jnp.float32)

    a1, a2 = _sc_build_adjacency(ei1, ea1, ei2, ea2)

    w1c = jnp.concatenate([W1[0] - W1[2], W1[1], W1[2]], axis=1)
    w4c = jnp.concatenate([W4[0] - W4[2], W4[1], W4[2]], axis=1)

    out1 = _tc_dense(x1, a1, w1c, b1, w4c, b4, Wc1, bc1, Wc2, bc2)
    out2 = _tc_dense(x2, a2, w1c, b1, w4c, b4, Wc1, bc1, Wc2, bc2)
    return out1, out2
